# baseline (speedup 1.0000x reference)
import functools

import jax
import jax.numpy as jnp
from jax import lax
from jax.experimental import pallas as pl
from jax.experimental.pallas import tpu as pltpu
from jax.experimental.pallas import tpu_sc as plsc

N = 10000
E = 320000
D = 128
G = 128
NC, NS = 2, 16
T = NC * NS
CHUNK = 128
NBUF = 2
NPH = 2
CPT = 80
HPT = CPT // NPH
EPT = CPT * CHUNK
NPAD = 10240
RPT = NPAD // NS

@functools.cache
def _mesh():
    return plsc.VectorSubcoreMesh(core_axis_name="c", subcore_axis_name="s")


def _fill(buf, value):
    @pl.loop(0, buf.shape[0])
    def _(i):
        @pl.loop(0, D // 16)
        def _(k):
            buf[i, pl.ds(k * 16, 16)] = jnp.full((16,), value, jnp.float32)


def _zero_shared(zbuf, shared, s):
    @pl.loop(0, RPT // CHUNK)
    def _(r):
        pltpu.sync_copy(zbuf, shared.at[pl.ds(s * RPT + r * CHUNK, CHUNK)])


@functools.cache
def _sc_hist_kernel():
    return pl.kernel(
        _sc_hist_body,
        mesh=_mesh(),
        out_type=jax.ShapeDtypeStruct((NC, NPAD, D), jnp.float32),
        scratch_types=[
            pltpu.VMEM((CPT, CHUNK), jnp.int32),
            pltpu.VMEM((CHUNK, D), jnp.float32),
            pltpu.VMEM_SHARED((NPAD, D), jnp.float32),
        ],
    )


def _sc_hist_body(dst_hbm, deg_hbm, dst_v, ones_v, hist_sh):
    c = lax.axis_index("c")
    s = lax.axis_index("s")
    wid = s * NC + c
    pltpu.sync_copy(dst_hbm.at[wid], dst_v)
    _fill(ones_v, 0.0)
    _zero_shared(ones_v, hist_sh, s)
    _fill(ones_v, 1.0)
    plsc.subcore_barrier()

    @pl.loop(0, CPT)
    def _(j):
        pltpu.sync_copy(ones_v, hist_sh.at[dst_v.at[j]], add=True)

    plsc.subcore_barrier()
    pltpu.sync_copy(hist_sh.at[pl.ds(s * RPT, RPT)],
                    deg_hbm.at[c].at[pl.ds(s * RPT, RPT)])


@functools.cache
def _sc_scatter_kernel():
    return pl.kernel(
        _sc_scatter_body,
        mesh=_mesh(),
        out_type=jax.ShapeDtypeStruct((NC, NPAD, D), jnp.float32),
        scratch_types=[
            pltpu.VMEM((CPT, CHUNK), jnp.int32),
            pltpu.VMEM((CPT, CHUNK), jnp.int32),
            pltpu.VMEM((CHUNK, D), jnp.float32),
            pltpu.VMEM_SHARED((NPAD, D), jnp.float32),
            pltpu.SemaphoreType.DMA,
        ],
    )


def _sc_scatter_body(y_hbm, src_hbm, dst_hbm, acc_hbm,
                     src_v, dst_v, rows_v, acc_sh, sem):
    c = lax.axis_index("c")
    s = lax.axis_index("s")
    wid = s * NC + c
    pltpu.sync_copy(src_hbm.at[wid], src_v)
    pltpu.sync_copy(dst_hbm.at[wid], dst_v)
    _fill(rows_v, 0.0)
    _zero_shared(rows_v, acc_sh, s)
    plsc.subcore_barrier()

    @pl.loop(0, CPT)
    def _(j):
        pltpu.async_copy(y_hbm.at[src_v.at[j]], rows_v, sem).wait()
        pltpu.sync_copy(rows_v, acc_sh.at[dst_v.at[j]], add=True)

    plsc.subcore_barrier()
    pltpu.sync_copy(acc_sh.at[pl.ds(s * RPT, RPT)],
                    acc_hbm.at[c].at[pl.ds(s * RPT, RPT)])


_RB = 1000


def _mm_body(x_ref, w_ref, o_ref):
    o_ref[...] = jnp.dot(x_ref[...], w_ref[...],
                         preferred_element_type=jnp.float32)


def _tc_matmul(x, w):
    return pl.pallas_call(
        _mm_body,
        grid=(N // _RB,),
        in_specs=[pl.BlockSpec((_RB, D), lambda i: (i, 0)),
                  pl.BlockSpec((D, D), lambda i: (0, 0))],
        out_specs=pl.BlockSpec((_RB, D), lambda i: (i, 0)),
        out_shape=jax.ShapeDtypeStruct((N, D), jnp.float32),
    )(x, w)


def _dis_y_body(degh_ref, xw_ref, dis_ref, y_ref):
    deg = degh_ref[0] + degh_ref[1] + 1.0
    dis = lax.rsqrt(deg)
    dis_ref[...] = dis[:, 0:16]
    y_ref[...] = xw_ref[...] * dis[:, 0:1]


def _tc_dis_y(degh, xw):
    return pl.pallas_call(
        _dis_y_body,
        grid=(N // _RB,),
        in_specs=[pl.BlockSpec((NC, _RB, D), lambda i: (0, i, 0)),
                  pl.BlockSpec((_RB, D), lambda i: (i, 0))],
        out_specs=[pl.BlockSpec((_RB, 16), lambda i: (i, 0)),
                   pl.BlockSpec((_RB, D), lambda i: (i, 0))],
        out_shape=[jax.ShapeDtypeStruct((N, 16), jnp.float32),
                   jax.ShapeDtypeStruct((N, D), jnp.float32)],
    )(degh, xw)


def _combine_mm_body(acc_ref, y_ref, dis_ref, b_ref, w_ref, y2_ref):
    dis = dis_ref[:, 0:1]
    h = dis * (acc_ref[0] + acc_ref[1] + y_ref[...]) + b_ref[...]
    h = jnp.maximum(h, 0.0)
    y2_ref[...] = jnp.dot(h, w_ref[...],
                          preferred_element_type=jnp.float32) * dis


def _tc_combine_mm(acc, y, dis, b, w):
    return pl.pallas_call(
        _combine_mm_body,
        grid=(N // _RB,),
        in_specs=[pl.BlockSpec((NC, _RB, D), lambda i: (0, i, 0)),
                  pl.BlockSpec((_RB, D), lambda i: (i, 0)),
                  pl.BlockSpec((_RB, 16), lambda i: (i, 0)),
                  pl.BlockSpec((1, D), lambda i: (0, 0)),
                  pl.BlockSpec((D, D), lambda i: (0, 0))],
        out_specs=pl.BlockSpec((_RB, D), lambda i: (i, 0)),
        out_shape=jax.ShapeDtypeStruct((N, D), jnp.float32),
    )(acc, y, dis, b, w)


def _pool_body(acc_ref, y2_ref, dis_ref, b_ref, batch_ref, wp_ref, bp_ref,
               out_ref, sums_sc, cnts_sc):
    i = pl.program_id(0)

    @pl.when(i == 0)
    def _():
        sums_sc[...] = jnp.zeros_like(sums_sc)
        cnts_sc[...] = jnp.zeros_like(cnts_sc)

    dis = dis_ref[:, 0:1]
    h2 = dis * (acc_ref[0] + acc_ref[1] + y2_ref[...]) + b_ref[...]
    gids = lax.broadcasted_iota(jnp.int32, (G, _RB), 0)
    oh = (batch_ref[0] == gids).astype(jnp.float32)
    sums_sc[...] += jnp.dot(oh, h2, preferred_element_type=jnp.float32)
    cnts_sc[...] += jnp.sum(oh, axis=1, keepdims=True)

    @pl.when(i == N // _RB - 1)
    def _():
        pooled = sums_sc[...] / jnp.maximum(cnts_sc[:, 0:1], 1.0)
        out_ref[...] = jnp.dot(pooled, wp_ref[...],
                               preferred_element_type=jnp.float32) + bp_ref[...]


def _tc_pool(acc, y2, dis, b, batch3d, wp_pad, bp_pad):
    return pl.pallas_call(
        _pool_body,
        grid=(N // _RB,),
        in_specs=[pl.BlockSpec((NC, _RB, D), lambda i: (0, i, 0)),
                  pl.BlockSpec((_RB, D), lambda i: (i, 0)),
                  pl.BlockSpec((_RB, 16), lambda i: (i, 0)),
                  pl.BlockSpec((1, D), lambda i: (0, 0)),
                  pl.BlockSpec((1, 1, _RB), lambda i: (i, 0, 0)),
                  pl.BlockSpec((D, D), lambda i: (0, 0)),
                  pl.BlockSpec((1, D), lambda i: (0, 0))],
        out_specs=pl.BlockSpec((G, D), lambda i: (0, 0)),
        out_shape=jax.ShapeDtypeStruct((G, D), jnp.float32),
        scratch_shapes=[pltpu.VMEM((G, D), jnp.float32),
                        pltpu.VMEM((G, 1), jnp.float32)],
    )(acc, y2, dis, b, batch3d, wp_pad, bp_pad)


def kernel(x, edge_index, batch, W1, b1, W2, b2, Wp, bp):
    src = edge_index[0].astype(jnp.int32)
    dst = edge_index[1].astype(jnp.int32)
    pad = T * EPT - E
    src_t = jnp.concatenate(
        [src, jnp.zeros((pad,), jnp.int32)]).reshape(T, CPT, CHUNK)
    dst_t = jnp.concatenate(
        [dst, jnp.full((pad,), N, jnp.int32)]).reshape(T, CPT, CHUNK)

    degh = _sc_hist_kernel()(dst_t)
    xw1 = _tc_matmul(x, W1)
    dis, y1 = _tc_dis_y(degh[:, :N], xw1)

    acc1 = _sc_scatter_kernel()(y1, src_t, dst_t)
    y2 = _tc_combine_mm(acc1[:, :N], y1, dis, b1.reshape(1, D), W2)

    acc2 = _sc_scatter_kernel()(y2, src_t, dst_t)
    batch3d = batch.astype(jnp.int32).reshape(N // _RB, 1, _RB)
    wp_pad = jnp.pad(Wp, ((0, 0), (0, D - Wp.shape[1])))
    bp_pad = jnp.pad(bp, (0, D - bp.shape[0])).reshape(1, D)
    out_pad = _tc_pool(acc2[:, :N], y2, dis, b2.reshape(1, D),
                       batch3d, wp_pad, bp_pad)
    return out_pad[:, :Wp.shape[1]]


# speedup vs baseline: 1.0856x; 1.0856x over previous
import functools

import jax
import jax.numpy as jnp
from jax import lax
from jax.experimental import pallas as pl
from jax.experimental.pallas import tpu as pltpu
from jax.experimental.pallas import tpu_sc as plsc

N = 10000
E = 320000
D = 128
G = 128
NC, NS = 2, 16
T = NC * NS
CHUNK = 128
NBUF = 2
NPH = 2
CPT = 80
HPT = CPT // NPH
EPT = CPT * CHUNK
NPAD = 10240
RPT = NPAD // NS

@functools.cache
def _mesh():
    return plsc.VectorSubcoreMesh(core_axis_name="c", subcore_axis_name="s")


def _fill(buf, value):
    @pl.loop(0, buf.shape[0])
    def _(i):
        @pl.loop(0, D // 16)
        def _(k):
            buf[i, pl.ds(k * 16, 16)] = jnp.full((16,), value, jnp.float32)


def _zero_shared(zbuf, shared, s):
    @pl.loop(0, RPT // CHUNK)
    def _(r):
        pltpu.sync_copy(zbuf, shared.at[pl.ds(s * RPT + r * CHUNK, CHUNK)])


@functools.cache
def _sc_hist_kernel():
    return pl.kernel(
        _sc_hist_body,
        mesh=_mesh(),
        out_type=jax.ShapeDtypeStruct((NC, NPAD, D), jnp.float32),
        scratch_types=[
            pltpu.VMEM((CPT, CHUNK), jnp.int32),
            pltpu.VMEM((CHUNK, D), jnp.float32),
            pltpu.VMEM_SHARED((NPAD, D), jnp.float32),
        ],
    )


def _sc_hist_body(dst_hbm, deg_hbm, dst_v, ones_v, hist_sh):
    c = lax.axis_index("c")
    s = lax.axis_index("s")
    wid = s * NC + c
    pltpu.sync_copy(dst_hbm.at[wid], dst_v)
    _fill(ones_v, 0.0)
    _zero_shared(ones_v, hist_sh, s)
    _fill(ones_v, 1.0)
    plsc.subcore_barrier()

    @pl.loop(0, CPT)
    def _(j):
        pltpu.sync_copy(ones_v, hist_sh.at[dst_v.at[j]], add=True)

    plsc.subcore_barrier()
    pltpu.sync_copy(hist_sh.at[pl.ds(s * RPT, RPT)],
                    deg_hbm.at[c].at[pl.ds(s * RPT, RPT)])


@functools.cache
def _sc_scatter_kernel():
    return pl.kernel(
        _sc_scatter_body,
        mesh=_mesh(),
        out_type=jax.ShapeDtypeStruct((NC, NPAD, D), jnp.float32),
        scratch_types=(
            [pltpu.VMEM((HPT, CHUNK), jnp.int32),
             pltpu.VMEM((HPT, CHUNK), jnp.int32)]
            + [pltpu.VMEM((CHUNK, D), jnp.float32)] * NBUF
            + [pltpu.VMEM_SHARED((NPAD, D), jnp.float32)]
            + [pltpu.SemaphoreType.DMA] * NBUF
        ),
    )


def _sc_scatter_body(y_hbm, src_hbm, dst_hbm, znd_hbm, acc_hbm,
                     src_v, dst_v, *rest):
    bufs = rest[:NBUF]
    acc_sh = rest[NBUF]
    sems = rest[NBUF + 1:]
    c = lax.axis_index("c")
    s = lax.axis_index("s")
    wid = s * NC + c
    pltpu.sync_copy(znd_hbm.at[pl.ds(s * RPT, RPT)],
                    acc_sh.at[pl.ds(s * RPT, RPT)])
    plsc.subcore_barrier()

    def gather(j, b):
        pltpu.make_async_copy(y_hbm.at[src_v.at[j]], bufs[b], sems[b]).start()

    def wait_scatter(j, b):
        pltpu.make_async_copy(y_hbm.at[src_v.at[j]], bufs[b], sems[b]).wait()
        pltpu.sync_copy(bufs[b], acc_sh.at[dst_v.at[j]], add=True)

    for ph in range(NPH):
        pltpu.sync_copy(src_hbm.at[wid].at[pl.ds(ph * HPT, HPT)], src_v)
        pltpu.sync_copy(dst_hbm.at[wid].at[pl.ds(ph * HPT, HPT)], dst_v)
        for b in range(NBUF):
            gather(b, b)

        @pl.loop(0, HPT // NBUF - 1)
        def _(p):
            base = p * NBUF
            for b in range(NBUF):
                wait_scatter(base + b, b)
                gather(base + NBUF + b, b)

        for b in range(NBUF):
            wait_scatter(HPT - NBUF + b, b)

    plsc.subcore_barrier()
    pltpu.sync_copy(acc_sh.at[pl.ds(s * RPT, RPT)],
                    acc_hbm.at[c].at[pl.ds(s * RPT, RPT)])


_RB = 1000


def _mm_body(x_ref, w_ref, o_ref):
    o_ref[...] = jnp.dot(x_ref[...], w_ref[...],
                         preferred_element_type=jnp.float32)


def _tc_matmul(x, w):
    return pl.pallas_call(
        _mm_body,
        grid=(N // _RB,),
        in_specs=[pl.BlockSpec((_RB, D), lambda i: (i, 0)),
                  pl.BlockSpec((D, D), lambda i: (0, 0))],
        out_specs=pl.BlockSpec((_RB, D), lambda i: (i, 0)),
        out_shape=jax.ShapeDtypeStruct((N, D), jnp.float32),
    )(x, w)


def _dis_y_body(degh_ref, xw_ref, dis_ref, y_ref):
    deg = degh_ref[0] + degh_ref[1] + 1.0
    dis = lax.rsqrt(deg)
    dis_ref[...] = dis[:, 0:16]
    y_ref[...] = xw_ref[...] * dis[:, 0:1]


def _tc_dis_y(degh, xw):
    return pl.pallas_call(
        _dis_y_body,
        grid=(N // _RB,),
        in_specs=[pl.BlockSpec((NC, _RB, D), lambda i: (0, i, 0)),
                  pl.BlockSpec((_RB, D), lambda i: (i, 0))],
        out_specs=[pl.BlockSpec((_RB, 16), lambda i: (i, 0)),
                   pl.BlockSpec((_RB, D), lambda i: (i, 0))],
        out_shape=[jax.ShapeDtypeStruct((N, 16), jnp.float32),
                   jax.ShapeDtypeStruct((N, D), jnp.float32)],
    )(degh, xw)


def _combine_mm_body(acc_ref, y_ref, dis_ref, b_ref, w_ref, y2_ref):
    dis = dis_ref[:, 0:1]
    h = dis * (acc_ref[0] + acc_ref[1] + y_ref[...]) + b_ref[...]
    h = jnp.maximum(h, 0.0)
    y2_ref[...] = jnp.dot(h, w_ref[...],
                          preferred_element_type=jnp.float32) * dis


def _tc_combine_mm(acc, y, dis, b, w):
    return pl.pallas_call(
        _combine_mm_body,
        grid=(N // _RB,),
        in_specs=[pl.BlockSpec((NC, _RB, D), lambda i: (0, i, 0)),
                  pl.BlockSpec((_RB, D), lambda i: (i, 0)),
                  pl.BlockSpec((_RB, 16), lambda i: (i, 0)),
                  pl.BlockSpec((1, D), lambda i: (0, 0)),
                  pl.BlockSpec((D, D), lambda i: (0, 0))],
        out_specs=pl.BlockSpec((_RB, D), lambda i: (i, 0)),
        out_shape=jax.ShapeDtypeStruct((N, D), jnp.float32),
    )(acc, y, dis, b, w)


def _pool_body(acc_ref, y2_ref, dis_ref, b_ref, batch_ref, wp_ref, bp_ref,
               out_ref, sums_sc, cnts_sc):
    i = pl.program_id(0)

    @pl.when(i == 0)
    def _():
        sums_sc[...] = jnp.zeros_like(sums_sc)
        cnts_sc[...] = jnp.zeros_like(cnts_sc)

    dis = dis_ref[:, 0:1]
    h2 = dis * (acc_ref[0] + acc_ref[1] + y2_ref[...]) + b_ref[...]
    gids = lax.broadcasted_iota(jnp.int32, (G, _RB), 0)
    oh = (batch_ref[0] == gids).astype(jnp.float32)
    sums_sc[...] += jnp.dot(oh, h2, preferred_element_type=jnp.float32)
    cnts_sc[...] += jnp.sum(oh, axis=1, keepdims=True)

    @pl.when(i == N // _RB - 1)
    def _():
        pooled = sums_sc[...] / jnp.maximum(cnts_sc[:, 0:1], 1.0)
        out_ref[...] = jnp.dot(pooled, wp_ref[...],
                               preferred_element_type=jnp.float32) + bp_ref[...]


def _tc_pool(acc, y2, dis, b, batch3d, wp_pad, bp_pad):
    return pl.pallas_call(
        _pool_body,
        grid=(N // _RB,),
        in_specs=[pl.BlockSpec((NC, _RB, D), lambda i: (0, i, 0)),
                  pl.BlockSpec((_RB, D), lambda i: (i, 0)),
                  pl.BlockSpec((_RB, 16), lambda i: (i, 0)),
                  pl.BlockSpec((1, D), lambda i: (0, 0)),
                  pl.BlockSpec((1, 1, _RB), lambda i: (i, 0, 0)),
                  pl.BlockSpec((D, D), lambda i: (0, 0)),
                  pl.BlockSpec((1, D), lambda i: (0, 0))],
        out_specs=pl.BlockSpec((G, D), lambda i: (0, 0)),
        out_shape=jax.ShapeDtypeStruct((G, D), jnp.float32),
        scratch_shapes=[pltpu.VMEM((G, D), jnp.float32),
                        pltpu.VMEM((G, 1), jnp.float32)],
    )(acc, y2, dis, b, batch3d, wp_pad, bp_pad)


def kernel(x, edge_index, batch, W1, b1, W2, b2, Wp, bp):
    src = edge_index[0].astype(jnp.int32)
    dst = edge_index[1].astype(jnp.int32)
    pad = T * EPT - E
    src_t = jnp.concatenate(
        [src, jnp.zeros((pad,), jnp.int32)]).reshape(T, CPT, CHUNK)
    dst_t = jnp.concatenate(
        [dst, jnp.full((pad,), N, jnp.int32)]).reshape(T, CPT, CHUNK)

    znd = jnp.zeros((NPAD, D), jnp.float32)
    degh = _sc_hist_kernel()(dst_t)
    xw1 = _tc_matmul(x, W1)
    dis, y1 = _tc_dis_y(degh[:, :N], xw1)

    acc1 = _sc_scatter_kernel()(y1, src_t, dst_t, znd)
    y2 = _tc_combine_mm(acc1[:, :N], y1, dis, b1.reshape(1, D), W2)

    acc2 = _sc_scatter_kernel()(y2, src_t, dst_t, znd)
    batch3d = batch.astype(jnp.int32).reshape(N // _RB, 1, _RB)
    wp_pad = jnp.pad(Wp, ((0, 0), (0, D - Wp.shape[1])))
    bp_pad = jnp.pad(bp, (0, D - bp.shape[0])).reshape(1, D)
    out_pad = _tc_pool(acc2[:, :N], y2, dis, b2.reshape(1, D),
                       batch3d, wp_pad, bp_pad)
    return out_pad[:, :Wp.shape[1]]


# speedup vs baseline: 1.0857x; 1.0001x over previous
import functools

import jax
import jax.numpy as jnp
from jax import lax
from jax.experimental import pallas as pl
from jax.experimental.pallas import tpu as pltpu
from jax.experimental.pallas import tpu_sc as plsc

N = 10000
E = 320000
D = 128
G = 128
NC, NS = 2, 16
T = NC * NS
CHUNK = 128
NBUF = 2
NPH = 2
CPT = 80
HPT = CPT // NPH
EPT = CPT * CHUNK
NPAD = 10240
RPT = NPAD // NS

@functools.cache
def _mesh():
    return plsc.VectorSubcoreMesh(core_axis_name="c", subcore_axis_name="s")


def _fill(buf, value):
    @pl.loop(0, buf.shape[0])
    def _(i):
        @pl.loop(0, D // 16)
        def _(k):
            buf[i, pl.ds(k * 16, 16)] = jnp.full((16,), value, jnp.float32)


def _zero_shared(zbuf, shared, s):
    @pl.loop(0, RPT // CHUNK)
    def _(r):
        pltpu.sync_copy(zbuf, shared.at[pl.ds(s * RPT + r * CHUNK, CHUNK)])


@functools.cache
def _sc_hist_kernel():
    return pl.kernel(
        _sc_hist_body,
        mesh=_mesh(),
        out_type=jax.ShapeDtypeStruct((NC, NPAD, D), jnp.float32),
        scratch_types=[
            pltpu.VMEM((CPT, CHUNK), jnp.int32),
            pltpu.VMEM((CHUNK, D), jnp.float32),
            pltpu.VMEM_SHARED((NPAD, D), jnp.float32),
        ],
    )


def _sc_hist_body(dst_hbm, deg_hbm, dst_v, ones_v, hist_sh):
    c = lax.axis_index("c")
    s = lax.axis_index("s")
    wid = s * NC + c
    pltpu.sync_copy(dst_hbm.at[wid], dst_v)
    _fill(ones_v, 0.0)
    _zero_shared(ones_v, hist_sh, s)
    _fill(ones_v, 1.0)
    plsc.subcore_barrier()

    @pl.loop(0, CPT)
    def _(j):
        pltpu.sync_copy(ones_v, hist_sh.at[dst_v.at[j]], add=True)

    plsc.subcore_barrier()
    pltpu.sync_copy(hist_sh.at[pl.ds(s * RPT, RPT)],
                    deg_hbm.at[c].at[pl.ds(s * RPT, RPT)])


@functools.cache
def _sc_scatter_kernel():
    return pl.kernel(
        _sc_scatter_body,
        mesh=_mesh(),
        out_type=jax.ShapeDtypeStruct((NC, NPAD, D), jnp.float32),
        scratch_types=(
            [pltpu.VMEM((HPT, CHUNK), jnp.int32),
             pltpu.VMEM((HPT, CHUNK), jnp.int32)]
            + [pltpu.VMEM((CHUNK, D), jnp.float32)] * NBUF
            + [pltpu.VMEM_SHARED((NPAD, D), jnp.float32)]
            + [pltpu.SemaphoreType.DMA] * NBUF
        ),
    )


def _sc_scatter_body(y_hbm, src_hbm, dst_hbm, znd_hbm, acc_hbm,
                     src_v, dst_v, *rest):
    bufs = rest[:NBUF]
    acc_sh = rest[NBUF]
    sems = rest[NBUF + 1:]
    c = lax.axis_index("c")
    s = lax.axis_index("s")
    wid = s * NC + c
    pltpu.sync_copy(znd_hbm.at[pl.ds(s * RPT, RPT)],
                    acc_sh.at[pl.ds(s * RPT, RPT)])
    plsc.subcore_barrier()

    def gather(j, b):
        pltpu.make_async_copy(y_hbm.at[src_v.at[j]], bufs[b], sems[b]).start()

    def wait_scatter(j, b):
        pltpu.make_async_copy(y_hbm.at[src_v.at[j]], bufs[b], sems[b]).wait()
        pltpu.sync_copy(bufs[b], acc_sh.at[dst_v.at[j]], add=True)

    for ph in range(NPH):
        pltpu.sync_copy(src_hbm.at[wid].at[pl.ds(ph * HPT, HPT)], src_v)
        pltpu.sync_copy(dst_hbm.at[wid].at[pl.ds(ph * HPT, HPT)], dst_v)
        for b in range(NBUF):
            gather(b, b)

        @pl.loop(0, HPT // NBUF - 1)
        def _(p):
            base = p * NBUF
            for b in range(NBUF):
                wait_scatter(base + b, b)
                gather(base + NBUF + b, b)

        for b in range(NBUF):
            wait_scatter(HPT - NBUF + b, b)

    plsc.subcore_barrier()
    pltpu.sync_copy(acc_sh.at[pl.ds(s * RPT, RPT)],
                    acc_hbm.at[c].at[pl.ds(s * RPT, RPT)])


_RB = 1000


def _mm_body(x_ref, w_ref, o_ref):
    o_ref[...] = jnp.dot(x_ref[...], w_ref[...],
                         preferred_element_type=jnp.float32)


def _tc_matmul(x, w):
    return pl.pallas_call(
        _mm_body,
        grid=(N // _RB,),
        in_specs=[pl.BlockSpec((_RB, D), lambda i: (i, 0)),
                  pl.BlockSpec((D, D), lambda i: (0, 0))],
        out_specs=pl.BlockSpec((_RB, D), lambda i: (i, 0)),
        out_shape=jax.ShapeDtypeStruct((N, D), jnp.float32),
    )(x, w)


def _dis_y_body(degh_ref, xw_ref, dis_ref, y_ref):
    deg = degh_ref[0] + degh_ref[1] + 1.0
    dis = lax.rsqrt(deg)
    dis_ref[...] = dis[:, 0:16]
    y_ref[...] = xw_ref[...] * dis[:, 0:1]


def _tc_dis_y(degh, xw):
    return pl.pallas_call(
        _dis_y_body,
        grid=(N // _RB,),
        in_specs=[pl.BlockSpec((NC, _RB, D), lambda i: (0, i, 0)),
                  pl.BlockSpec((_RB, D), lambda i: (i, 0))],
        out_specs=[pl.BlockSpec((_RB, 16), lambda i: (i, 0)),
                   pl.BlockSpec((_RB, D), lambda i: (i, 0))],
        out_shape=[jax.ShapeDtypeStruct((N, 16), jnp.float32),
                   jax.ShapeDtypeStruct((N, D), jnp.float32)],
    )(degh, xw)


def _combine_mm_body(acc_ref, y_ref, dis_ref, b_ref, w_ref, y2_ref):
    dis = dis_ref[:, 0:1]
    h = dis * (acc_ref[0] + acc_ref[1] + y_ref[...]) + b_ref[...]
    h = jnp.maximum(h, 0.0)
    y2_ref[...] = jnp.dot(h, w_ref[...],
                          preferred_element_type=jnp.float32) * dis


def _tc_combine_mm(acc, y, dis, b, w):
    return pl.pallas_call(
        _combine_mm_body,
        grid=(N // _RB,),
        in_specs=[pl.BlockSpec((NC, _RB, D), lambda i: (0, i, 0)),
                  pl.BlockSpec((_RB, D), lambda i: (i, 0)),
                  pl.BlockSpec((_RB, 16), lambda i: (i, 0)),
                  pl.BlockSpec((1, D), lambda i: (0, 0)),
                  pl.BlockSpec((D, D), lambda i: (0, 0))],
        out_specs=pl.BlockSpec((_RB, D), lambda i: (i, 0)),
        out_shape=jax.ShapeDtypeStruct((N, D), jnp.float32),
    )(acc, y, dis, b, w)


def _pool_body(acc_ref, y2_ref, dis_ref, b_ref, batch_ref, wp_ref, bp_ref,
               out_ref, sums_sc, cnts_sc):
    i = pl.program_id(0)

    @pl.when(i == 0)
    def _():
        sums_sc[...] = jnp.zeros_like(sums_sc)
        cnts_sc[...] = jnp.zeros_like(cnts_sc)

    dis = dis_ref[:, 0:1]
    h2 = dis * (acc_ref[0] + acc_ref[1] + y2_ref[...]) + b_ref[...]
    gids = lax.broadcasted_iota(jnp.int32, (G, _RB), 0)
    oh = (batch_ref[0] == gids).astype(jnp.float32)
    sums_sc[...] += jnp.dot(oh, h2, preferred_element_type=jnp.float32)
    cnts_sc[...] += jnp.sum(oh, axis=1, keepdims=True)

    @pl.when(i == N // _RB - 1)
    def _():
        pooled = sums_sc[...] / jnp.maximum(cnts_sc[:, 0:1], 1.0)
        out_ref[...] = jnp.dot(pooled, wp_ref[...],
                               preferred_element_type=jnp.float32) + bp_ref[...]


def _tc_pool(acc, y2, dis, b, batch3d, wp_pad, bp_pad):
    return pl.pallas_call(
        _pool_body,
        grid=(N // _RB,),
        in_specs=[pl.BlockSpec((NC, _RB, D), lambda i: (0, i, 0)),
                  pl.BlockSpec((_RB, D), lambda i: (i, 0)),
                  pl.BlockSpec((_RB, 16), lambda i: (i, 0)),
                  pl.BlockSpec((1, D), lambda i: (0, 0)),
                  pl.BlockSpec((1, 1, _RB), lambda i: (i, 0, 0)),
                  pl.BlockSpec((D, D), lambda i: (0, 0)),
                  pl.BlockSpec((1, D), lambda i: (0, 0))],
        out_specs=pl.BlockSpec((G, D), lambda i: (0, 0)),
        out_shape=jax.ShapeDtypeStruct((G, D), jnp.float32),
        scratch_shapes=[pltpu.VMEM((G, D), jnp.float32),
                        pltpu.VMEM((G, 1), jnp.float32)],
    )(acc, y2, dis, b, batch3d, wp_pad, bp_pad)


def kernel(x, edge_index, batch, W1, b1, W2, b2, Wp, bp):
    src = edge_index[0].astype(jnp.int32)
    dst = edge_index[1].astype(jnp.int32)
    pad = T * EPT - E
    pad_dst = N + (jnp.arange(pad, dtype=jnp.int32) % (NPAD - N))
    src_t = jnp.concatenate(
        [src, jnp.zeros((pad,), jnp.int32)]).reshape(T, CPT, CHUNK)
    dst_t = jnp.concatenate([dst, pad_dst]).reshape(T, CPT, CHUNK)

    znd = jnp.zeros((NPAD, D), jnp.float32)
    degh = _sc_hist_kernel()(dst_t)
    xw1 = _tc_matmul(x, W1)
    dis, y1 = _tc_dis_y(degh[:, :N], xw1)

    acc1 = _sc_scatter_kernel()(y1, src_t, dst_t, znd)
    y2 = _tc_combine_mm(acc1[:, :N], y1, dis, b1.reshape(1, D), W2)

    acc2 = _sc_scatter_kernel()(y2, src_t, dst_t, znd)
    batch3d = batch.astype(jnp.int32).reshape(N // _RB, 1, _RB)
    wp_pad = jnp.pad(Wp, ((0, 0), (0, D - Wp.shape[1])))
    bp_pad = jnp.pad(bp, (0, D - bp.shape[0])).reshape(1, D)
    out_pad = _tc_pool(acc2[:, :N], y2, dis, b2.reshape(1, D),
                       batch3d, wp_pad, bp_pad)
    return out_pad[:, :Wp.shape[1]]


# speedup vs baseline: 1.4904x; 1.3728x over previous
import functools

import jax
import jax.numpy as jnp
from jax import lax
from jax.experimental import pallas as pl
from jax.experimental.pallas import tpu as pltpu
from jax.experimental.pallas import tpu_sc as plsc

N = 10000
E = 320000
D = 128
G = 128
NC, NS = 2, 16
T = NC * NS
CHUNK = 128
CPT = -(-E // (T * CHUNK))
EPT = CPT * CHUNK
NPAD = 10240
RPT = NPAD // NS

@functools.cache
def _mesh():
    return plsc.VectorSubcoreMesh(core_axis_name="c", subcore_axis_name="s")


def _fill(buf, value):
    @pl.loop(0, buf.shape[0])
    def _(i):
        @pl.loop(0, D // 16)
        def _(k):
            buf[i, pl.ds(k * 16, 16)] = jnp.full((16,), value, jnp.float32)


def _zero_shared(zbuf, shared, s):
    @pl.loop(0, RPT // CHUNK)
    def _(r):
        pltpu.sync_copy(zbuf, shared.at[pl.ds(s * RPT + r * CHUNK, CHUNK)])


@functools.cache
def _sc_hist_kernel():
    return pl.kernel(
        _sc_hist_body,
        mesh=_mesh(),
        out_type=jax.ShapeDtypeStruct((NC, NPAD, D), jnp.float32),
        scratch_types=[
            pltpu.VMEM((CPT, CHUNK), jnp.int32),
            pltpu.VMEM((CHUNK, D), jnp.float32),
            pltpu.VMEM_SHARED((NPAD, D), jnp.float32),
        ],
    )


def _sc_hist_body(dst_hbm, deg_hbm, dst_v, ones_v, hist_sh):
    c = lax.axis_index("c")
    s = lax.axis_index("s")
    wid = s * NC + c
    pltpu.sync_copy(dst_hbm.at[wid], dst_v)
    _fill(ones_v, 0.0)
    _zero_shared(ones_v, hist_sh, s)
    _fill(ones_v, 1.0)
    plsc.subcore_barrier()

    @pl.loop(0, CPT)
    def _(j):
        pltpu.sync_copy(ones_v, hist_sh.at[dst_v.at[j]], add=True)

    plsc.subcore_barrier()
    pltpu.sync_copy(hist_sh.at[pl.ds(s * RPT, RPT)],
                    deg_hbm.at[c].at[pl.ds(s * RPT, RPT)])


@functools.cache
def _sc_scatter_kernel():
    return pl.kernel(
        _sc_scatter_body,
        mesh=_mesh(),
        out_type=jax.ShapeDtypeStruct((NC, NPAD, D), jnp.float32),
        scratch_types=[
            pltpu.VMEM((CPT, CHUNK), jnp.int32),
            pltpu.VMEM((CPT, CHUNK), jnp.int32),
            pltpu.VMEM((CHUNK, D), jnp.float32),
            pltpu.VMEM_SHARED((NPAD, D), jnp.float32),
            pltpu.SemaphoreType.DMA,
        ],
    )


def _sc_scatter_body(y_hbm, src_hbm, dst_hbm, znd_hbm, acc_hbm,
                     src_v, dst_v, rows_v, acc_sh, sem):
    c = lax.axis_index("c")
    s = lax.axis_index("s")
    wid = s * NC + c
    pltpu.sync_copy(src_hbm.at[wid], src_v)
    pltpu.sync_copy(dst_hbm.at[wid], dst_v)
    pltpu.sync_copy(znd_hbm.at[pl.ds(s * RPT, RPT)],
                    acc_sh.at[pl.ds(s * RPT, RPT)])
    plsc.subcore_barrier()

    @pl.loop(0, CPT)
    def _(j):
        pltpu.async_copy(y_hbm.at[src_v.at[j]], rows_v, sem).wait()
        pltpu.sync_copy(rows_v, acc_sh.at[dst_v.at[j]], add=True)

    plsc.subcore_barrier()
    pltpu.sync_copy(acc_sh.at[pl.ds(s * RPT, RPT)],
                    acc_hbm.at[c].at[pl.ds(s * RPT, RPT)])


_RB = 1000


def _mm_body(x_ref, w_ref, o_ref):
    o_ref[...] = jnp.dot(x_ref[...], w_ref[...],
                         preferred_element_type=jnp.float32)


def _tc_matmul(x, w):
    return pl.pallas_call(
        _mm_body,
        grid=(N // _RB,),
        in_specs=[pl.BlockSpec((_RB, D), lambda i: (i, 0)),
                  pl.BlockSpec((D, D), lambda i: (0, 0))],
        out_specs=pl.BlockSpec((_RB, D), lambda i: (i, 0)),
        out_shape=jax.ShapeDtypeStruct((N, D), jnp.float32),
    )(x, w)


def _dis_y_body(degh_ref, xw_ref, dis_ref, y_ref):
    deg = degh_ref[0] + degh_ref[1] + 1.0
    dis = lax.rsqrt(deg)
    dis_ref[...] = dis[:, 0:16]
    y_ref[...] = xw_ref[...] * dis[:, 0:1]


def _tc_dis_y(degh, xw):
    return pl.pallas_call(
        _dis_y_body,
        grid=(N // _RB,),
        in_specs=[pl.BlockSpec((NC, _RB, D), lambda i: (0, i, 0)),
                  pl.BlockSpec((_RB, D), lambda i: (i, 0))],
        out_specs=[pl.BlockSpec((_RB, 16), lambda i: (i, 0)),
                   pl.BlockSpec((_RB, D), lambda i: (i, 0))],
        out_shape=[jax.ShapeDtypeStruct((N, 16), jnp.float32),
                   jax.ShapeDtypeStruct((N, D), jnp.float32)],
    )(degh, xw)


def _combine_mm_body(acc_ref, y_ref, dis_ref, b_ref, w_ref, y2_ref):
    dis = dis_ref[:, 0:1]
    h = dis * (acc_ref[0] + acc_ref[1] + y_ref[...]) + b_ref[...]
    h = jnp.maximum(h, 0.0)
    y2_ref[...] = jnp.dot(h, w_ref[...],
                          preferred_element_type=jnp.float32) * dis


def _tc_combine_mm(acc, y, dis, b, w):
    return pl.pallas_call(
        _combine_mm_body,
        grid=(N // _RB,),
        in_specs=[pl.BlockSpec((NC, _RB, D), lambda i: (0, i, 0)),
                  pl.BlockSpec((_RB, D), lambda i: (i, 0)),
                  pl.BlockSpec((_RB, 16), lambda i: (i, 0)),
                  pl.BlockSpec((1, D), lambda i: (0, 0)),
                  pl.BlockSpec((D, D), lambda i: (0, 0))],
        out_specs=pl.BlockSpec((_RB, D), lambda i: (i, 0)),
        out_shape=jax.ShapeDtypeStruct((N, D), jnp.float32),
    )(acc, y, dis, b, w)


def _pool_body(acc_ref, y2_ref, dis_ref, b_ref, batch_ref, wp_ref, bp_ref,
               out_ref, sums_sc, cnts_sc):
    i = pl.program_id(0)

    @pl.when(i == 0)
    def _():
        sums_sc[...] = jnp.zeros_like(sums_sc)
        cnts_sc[...] = jnp.zeros_like(cnts_sc)

    dis = dis_ref[:, 0:1]
    h2 = dis * (acc_ref[0] + acc_ref[1] + y2_ref[...]) + b_ref[...]
    gids = lax.broadcasted_iota(jnp.int32, (G, _RB), 0)
    oh = (batch_ref[0] == gids).astype(jnp.float32)
    sums_sc[...] += jnp.dot(oh, h2, preferred_element_type=jnp.float32)
    cnts_sc[...] += jnp.sum(oh, axis=1, keepdims=True)

    @pl.when(i == N // _RB - 1)
    def _():
        pooled = sums_sc[...] / jnp.maximum(cnts_sc[:, 0:1], 1.0)
        out_ref[...] = jnp.dot(pooled, wp_ref[...],
                               preferred_element_type=jnp.float32) + bp_ref[...]


def _tc_pool(acc, y2, dis, b, batch3d, wp_pad, bp_pad):
    return pl.pallas_call(
        _pool_body,
        grid=(N // _RB,),
        in_specs=[pl.BlockSpec((NC, _RB, D), lambda i: (0, i, 0)),
                  pl.BlockSpec((_RB, D), lambda i: (i, 0)),
                  pl.BlockSpec((_RB, 16), lambda i: (i, 0)),
                  pl.BlockSpec((1, D), lambda i: (0, 0)),
                  pl.BlockSpec((1, 1, _RB), lambda i: (i, 0, 0)),
                  pl.BlockSpec((D, D), lambda i: (0, 0)),
                  pl.BlockSpec((1, D), lambda i: (0, 0))],
        out_specs=pl.BlockSpec((G, D), lambda i: (0, 0)),
        out_shape=jax.ShapeDtypeStruct((G, D), jnp.float32),
        scratch_shapes=[pltpu.VMEM((G, D), jnp.float32),
                        pltpu.VMEM((G, 1), jnp.float32)],
    )(acc, y2, dis, b, batch3d, wp_pad, bp_pad)


def kernel(x, edge_index, batch, W1, b1, W2, b2, Wp, bp):
    src = edge_index[0].astype(jnp.int32)
    dst = edge_index[1].astype(jnp.int32)
    pad = T * EPT - E
    pad_dst = N + (jnp.arange(pad, dtype=jnp.int32) % (NPAD - N))
    src_t = jnp.concatenate(
        [src, jnp.zeros((pad,), jnp.int32)]).reshape(T, CPT, CHUNK)
    dst_t = jnp.concatenate([dst, pad_dst]).reshape(T, CPT, CHUNK)

    znd = jnp.zeros((NPAD, D), jnp.float32)
    degh = _sc_hist_kernel()(dst_t)
    xw1 = _tc_matmul(x, W1)
    dis, y1 = _tc_dis_y(degh[:, :N], xw1)

    acc1 = _sc_scatter_kernel()(y1, src_t, dst_t, znd)
    y2 = _tc_combine_mm(acc1[:, :N], y1, dis, b1.reshape(1, D), W2)

    acc2 = _sc_scatter_kernel()(y2, src_t, dst_t, znd)
    batch3d = batch.astype(jnp.int32).reshape(N // _RB, 1, _RB)
    wp_pad = jnp.pad(Wp, ((0, 0), (0, D - Wp.shape[1])))
    bp_pad = jnp.pad(bp, (0, D - bp.shape[0])).reshape(1, D)
    out_pad = _tc_pool(acc2[:, :N], y2, dis, b2.reshape(1, D),
                       batch3d, wp_pad, bp_pad)
    return out_pad[:, :Wp.shape[1]]


# speedup vs baseline: 1.5770x; 1.0581x over previous
import functools

import jax
import jax.numpy as jnp
from jax import lax
from jax.experimental import pallas as pl
from jax.experimental.pallas import tpu as pltpu
from jax.experimental.pallas import tpu_sc as plsc

N = 10000
E = 320000
D = 128
G = 128
NC, NS = 2, 16
T = NC * NS
CHUNK = 128
CPT = -(-E // (T * CHUNK))
EPT = CPT * CHUNK
NPAD = 10240
RPT = NPAD // NS

@functools.cache
def _mesh():
    return plsc.VectorSubcoreMesh(core_axis_name="c", subcore_axis_name="s")


def _fill(buf, value):
    @pl.loop(0, buf.shape[0])
    def _(i):
        @pl.loop(0, D // 16)
        def _(k):
            buf[i, pl.ds(k * 16, 16)] = jnp.full((16,), value, jnp.float32)


def _zero_shared(zbuf, shared, s):
    @pl.loop(0, RPT // CHUNK)
    def _(r):
        pltpu.sync_copy(zbuf, shared.at[pl.ds(s * RPT + r * CHUNK, CHUNK)])


@functools.cache
def _sc_hist_kernel():
    return pl.kernel(
        _sc_hist_body,
        mesh=_mesh(),
        out_type=jax.ShapeDtypeStruct((NC, NPAD, D), jnp.float32),
        scratch_types=[
            pltpu.VMEM((CPT, CHUNK), jnp.int32),
            pltpu.VMEM((CHUNK, D), jnp.float32),
            pltpu.VMEM_SHARED((NPAD, D), jnp.float32),
        ],
    )


def _sc_hist_body(dst_hbm, deg_hbm, dst_v, ones_v, hist_sh):
    c = lax.axis_index("c")
    s = lax.axis_index("s")
    wid = s * NC + c
    pltpu.sync_copy(dst_hbm.at[wid], dst_v)
    _fill(ones_v, 0.0)
    _zero_shared(ones_v, hist_sh, s)
    _fill(ones_v, 1.0)
    plsc.subcore_barrier()

    @pl.loop(0, CPT)
    def _(j):
        pltpu.sync_copy(ones_v, hist_sh.at[dst_v.at[j]], add=True)

    plsc.subcore_barrier()
    pltpu.sync_copy(hist_sh.at[pl.ds(s * RPT, RPT)],
                    deg_hbm.at[c].at[pl.ds(s * RPT, RPT)])


@functools.cache
def _sc_scatter_kernel():
    return pl.kernel(
        _sc_scatter_body,
        mesh=_mesh(),
        out_type=jax.ShapeDtypeStruct((NC, NPAD, D), jnp.float32),
        scratch_types=[
            pltpu.VMEM((CPT, CHUNK), jnp.int32),
            pltpu.VMEM((CPT, CHUNK), jnp.int32),
            pltpu.VMEM((CHUNK, D), jnp.float32),
            pltpu.VMEM_SHARED((NPAD, D), jnp.float32),
            pltpu.SemaphoreType.DMA,
        ],
    )


def _sc_scatter_body(y_hbm, src_hbm, dst_hbm, znd_hbm, acc_hbm,
                     src_v, dst_v, rows_v, acc_sh, sem):
    c = lax.axis_index("c")
    s = lax.axis_index("s")
    wid = s * NC + c
    pltpu.sync_copy(src_hbm.at[wid], src_v)
    pltpu.sync_copy(dst_hbm.at[wid], dst_v)
    pltpu.sync_copy(znd_hbm.at[pl.ds(s * RPT, RPT)],
                    acc_sh.at[pl.ds(s * RPT, RPT)])
    plsc.subcore_barrier()

    @pl.loop(0, CPT)
    def _(j):
        pltpu.async_copy(y_hbm.at[src_v.at[j]], rows_v, sem).wait()
        pltpu.sync_copy(rows_v, acc_sh.at[dst_v.at[j]], add=True)

    plsc.subcore_barrier()
    pltpu.sync_copy(acc_sh.at[pl.ds(s * RPT, RPT)],
                    acc_hbm.at[c].at[pl.ds(s * RPT, RPT)])


_RB = 1000


def _mm_body(x_ref, w_ref, o_ref):
    o_ref[...] = jnp.dot(x_ref[...], w_ref[...],
                         preferred_element_type=jnp.float32)


def _tc_matmul(x, w):
    return pl.pallas_call(
        _mm_body,
        grid=(N // _RB,),
        in_specs=[pl.BlockSpec((_RB, D), lambda i: (i, 0)),
                  pl.BlockSpec((D, D), lambda i: (0, 0))],
        out_specs=pl.BlockSpec((_RB, D), lambda i: (i, 0)),
        out_shape=jax.ShapeDtypeStruct((N, D), jnp.float32),
    )(x, w)


def _dis_y_body(degh_ref, xw_ref, dis_ref, y_ref):
    deg = degh_ref[0] + degh_ref[1] + 1.0
    dis = lax.rsqrt(deg)
    dis_ref[...] = dis[:, 0:16]
    y_ref[...] = xw_ref[...] * dis[:, 0:1]


def _tc_dis_y(degh, xw):
    return pl.pallas_call(
        _dis_y_body,
        grid=(N // _RB,),
        in_specs=[pl.BlockSpec((NC, _RB, D), lambda i: (0, i, 0)),
                  pl.BlockSpec((_RB, D), lambda i: (i, 0))],
        out_specs=[pl.BlockSpec((_RB, 16), lambda i: (i, 0)),
                   pl.BlockSpec((_RB, D), lambda i: (i, 0))],
        out_shape=[jax.ShapeDtypeStruct((N, 16), jnp.float32),
                   jax.ShapeDtypeStruct((N, D), jnp.float32)],
    )(degh, xw)


def _combine_mm_body(acc_ref, y_ref, dis_ref, b_ref, w_ref, y2_ref):
    dis = dis_ref[:, 0:1]
    h = dis * (acc_ref[0] + acc_ref[1] + y_ref[...]) + b_ref[...]
    h = jnp.maximum(h, 0.0)
    y2_ref[...] = jnp.dot(h, w_ref[...],
                          preferred_element_type=jnp.float32) * dis


def _tc_combine_mm(acc, y, dis, b, w):
    return pl.pallas_call(
        _combine_mm_body,
        grid=(N // _RB,),
        in_specs=[pl.BlockSpec((NC, _RB, D), lambda i: (0, i, 0)),
                  pl.BlockSpec((_RB, D), lambda i: (i, 0)),
                  pl.BlockSpec((_RB, 16), lambda i: (i, 0)),
                  pl.BlockSpec((1, D), lambda i: (0, 0)),
                  pl.BlockSpec((D, D), lambda i: (0, 0))],
        out_specs=pl.BlockSpec((_RB, D), lambda i: (i, 0)),
        out_shape=jax.ShapeDtypeStruct((N, D), jnp.float32),
    )(acc, y, dis, b, w)


def _pool_body(acc_ref, y2_ref, dis_ref, b_ref, batch_ref, wp_ref, bp_ref,
               out_ref, sums_sc, cnts_sc):
    i = pl.program_id(0)

    @pl.when(i == 0)
    def _():
        sums_sc[...] = jnp.zeros_like(sums_sc)
        cnts_sc[...] = jnp.zeros_like(cnts_sc)

    dis = dis_ref[:, 0:1]
    h2 = dis * (acc_ref[0] + acc_ref[1] + y2_ref[...]) + b_ref[...]
    gids = lax.broadcasted_iota(jnp.int32, (G, _RB), 0)
    oh = (batch_ref[0] == gids).astype(jnp.float32)
    sums_sc[...] += jnp.dot(oh, h2, preferred_element_type=jnp.float32)
    cnts_sc[...] += jnp.sum(oh, axis=1, keepdims=True)

    @pl.when(i == N // _RB - 1)
    def _():
        pooled = sums_sc[...] / jnp.maximum(cnts_sc[:, 0:1], 1.0)
        out_ref[...] = jnp.dot(pooled, wp_ref[...],
                               preferred_element_type=jnp.float32) + bp_ref[...]


def _tc_pool(acc, y2, dis, b, batch3d, wp_pad, bp_pad):
    return pl.pallas_call(
        _pool_body,
        grid=(N // _RB,),
        in_specs=[pl.BlockSpec((NC, _RB, D), lambda i: (0, i, 0)),
                  pl.BlockSpec((_RB, D), lambda i: (i, 0)),
                  pl.BlockSpec((_RB, 16), lambda i: (i, 0)),
                  pl.BlockSpec((1, D), lambda i: (0, 0)),
                  pl.BlockSpec((1, 1, _RB), lambda i: (i, 0, 0)),
                  pl.BlockSpec((D, D), lambda i: (0, 0)),
                  pl.BlockSpec((1, D), lambda i: (0, 0))],
        out_specs=pl.BlockSpec((G, D), lambda i: (0, 0)),
        out_shape=jax.ShapeDtypeStruct((G, D), jnp.float32),
        scratch_shapes=[pltpu.VMEM((G, D), jnp.float32),
                        pltpu.VMEM((G, 1), jnp.float32)],
    )(acc, y2, dis, b, batch3d, wp_pad, bp_pad)


def kernel(x, edge_index, batch, W1, b1, W2, b2, Wp, bp):
    src = edge_index[0].astype(jnp.int32)
    dst = edge_index[1].astype(jnp.int32)
    pad = T * EPT - E
    pad_dst = N + (jnp.arange(pad, dtype=jnp.int32) % (NPAD - N))
    src_t = jnp.concatenate(
        [src, jnp.zeros((pad,), jnp.int32)]).reshape(T, CPT, CHUNK)
    dst_t = jnp.concatenate([dst, pad_dst]).reshape(T, CPT, CHUNK)

    znd = jnp.zeros((NPAD, D), jnp.float32)
    degh = _sc_hist_kernel()(dst_t)
    xw1 = _tc_matmul(x, W1)
    dis, y1 = _tc_dis_y(degh, xw1)

    acc1 = _sc_scatter_kernel()(y1, src_t, dst_t, znd)
    y2 = _tc_combine_mm(acc1, y1, dis, b1.reshape(1, D), W2)

    acc2 = _sc_scatter_kernel()(y2, src_t, dst_t, znd)
    batch3d = batch.astype(jnp.int32).reshape(N // _RB, 1, _RB)
    wp_pad = jnp.pad(Wp, ((0, 0), (0, D - Wp.shape[1])))
    bp_pad = jnp.pad(bp, (0, D - bp.shape[0])).reshape(1, D)
    out_pad = _tc_pool(acc2, y2, dis, b2.reshape(1, D),
                       batch3d, wp_pad, bp_pad)
    return out_pad[:, :Wp.shape[1]]


# speedup vs baseline: 2.4885x; 1.5780x over previous
import functools

import jax
import jax.numpy as jnp
from jax import lax
from jax.experimental import pallas as pl
from jax.experimental.pallas import tpu as pltpu
from jax.experimental.pallas import tpu_sc as plsc

N = 10000
E = 320000
D = 128
G = 128
NC, NS = 2, 16
T = NC * NS
CHUNK = 128
CPT = -(-E // (T * CHUNK))
EPT = CPT * CHUNK
NPAD = 10240
RPT = NPAD // NS

@functools.cache
def _mesh():
    return plsc.VectorSubcoreMesh(core_axis_name="c", subcore_axis_name="s")


def _fill(buf, value):
    @pl.loop(0, buf.shape[0])
    def _(i):
        @pl.loop(0, D // 16)
        def _(k):
            buf[i, pl.ds(k * 16, 16)] = jnp.full((16,), value, jnp.float32)


def _zero_shared(zbuf, shared, s):
    @pl.loop(0, RPT // CHUNK)
    def _(r):
        pltpu.sync_copy(zbuf, shared.at[pl.ds(s * RPT + r * CHUNK, CHUNK)])


@functools.cache
def _sc_hist_kernel():
    return pl.kernel(
        _sc_hist_body,
        mesh=_mesh(),
        out_type=jax.ShapeDtypeStruct((NC, NPAD, D), jnp.float32),
        scratch_types=[
            pltpu.VMEM((CPT, CHUNK), jnp.int32),
            pltpu.VMEM((CHUNK, D), jnp.float32),
            pltpu.VMEM_SHARED((NPAD, D), jnp.float32),
        ],
    )


def _sc_hist_body(dst_hbm, deg_hbm, dst_v, ones_v, hist_sh):
    c = lax.axis_index("c")
    s = lax.axis_index("s")
    wid = s * NC + c
    pltpu.sync_copy(dst_hbm.at[wid], dst_v)
    _fill(ones_v, 0.0)
    _zero_shared(ones_v, hist_sh, s)
    _fill(ones_v, 1.0)
    plsc.subcore_barrier()

    @pl.loop(0, CPT)
    def _(j):
        pltpu.sync_copy(ones_v, hist_sh.at[dst_v.at[j]], add=True)

    plsc.subcore_barrier()
    pltpu.sync_copy(hist_sh.at[pl.ds(s * RPT, RPT)],
                    deg_hbm.at[c].at[pl.ds(s * RPT, RPT)])


@functools.cache
def _sc_scatter_kernel():
    return pl.kernel(
        _sc_scatter_body,
        mesh=_mesh(),
        out_type=jax.ShapeDtypeStruct((NC, NPAD, D), jnp.float32),
        scratch_types=[
            pltpu.VMEM((CPT, CHUNK), jnp.int32),
            pltpu.VMEM((CPT, CHUNK), jnp.int32),
            pltpu.VMEM((CHUNK, D), jnp.float32),
            pltpu.VMEM_SHARED((NPAD, D), jnp.float32),
            pltpu.SemaphoreType.DMA,
        ],
    )


def _sc_scatter_body(y_hbm, src_hbm, dst_hbm, znd_hbm, acc_hbm,
                     src_v, dst_v, rows_v, acc_sh, sem):
    c = lax.axis_index("c")
    s = lax.axis_index("s")
    wid = s * NC + c
    pltpu.sync_copy(src_hbm.at[wid], src_v)
    pltpu.sync_copy(dst_hbm.at[wid], dst_v)
    pltpu.sync_copy(znd_hbm.at[pl.ds(s * RPT, RPT)],
                    acc_sh.at[pl.ds(s * RPT, RPT)])
    plsc.subcore_barrier()

    @pl.loop(0, CPT)
    def _(j):
        pltpu.async_copy(y_hbm.at[src_v.at[j]], rows_v, sem).wait()
        pltpu.sync_copy(rows_v, acc_sh.at[dst_v.at[j]], add=True)

    plsc.subcore_barrier()
    pltpu.sync_copy(acc_sh.at[pl.ds(s * RPT, RPT)],
                    acc_hbm.at[c].at[pl.ds(s * RPT, RPT)])


_RB = 1000


def _mm_body(x_ref, w_ref, o_ref):
    o_ref[...] = jnp.dot(x_ref[...], w_ref[...],
                         preferred_element_type=jnp.float32)


def _tc_matmul(x, w):
    return pl.pallas_call(
        _mm_body,
        grid=(N // _RB,),
        in_specs=[pl.BlockSpec((_RB, D), lambda i: (i, 0)),
                  pl.BlockSpec((D, D), lambda i: (0, 0))],
        out_specs=pl.BlockSpec((_RB, D), lambda i: (i, 0)),
        out_shape=jax.ShapeDtypeStruct((N, D), jnp.float32),
    )(x, w)


def _dis_y_body(degh_ref, xw_ref, dis_ref, y_ref):
    deg = degh_ref[0] + degh_ref[1] + 1.0
    dis = lax.rsqrt(deg)
    dis_ref[...] = dis[:, 0:16]
    y_ref[...] = xw_ref[...] * dis[:, 0:1]


def _tc_dis_y(degh, xw):
    return pl.pallas_call(
        _dis_y_body,
        grid=(N // _RB,),
        in_specs=[pl.BlockSpec((NC, _RB, D), lambda i: (0, i, 0)),
                  pl.BlockSpec((_RB, D), lambda i: (i, 0))],
        out_specs=[pl.BlockSpec((_RB, 16), lambda i: (i, 0)),
                   pl.BlockSpec((_RB, D), lambda i: (i, 0))],
        out_shape=[jax.ShapeDtypeStruct((N, 16), jnp.float32),
                   jax.ShapeDtypeStruct((N, D), jnp.float32)],
    )(degh, xw)


def _combine_mm_body(acc_ref, y_ref, dis_ref, b_ref, w_ref, y2_ref):
    dis = dis_ref[:, 0:1]
    h = dis * (acc_ref[0] + acc_ref[1] + y_ref[...]) + b_ref[...]
    h = jnp.maximum(h, 0.0)
    y2_ref[...] = jnp.dot(h, w_ref[...],
                          preferred_element_type=jnp.float32) * dis


def _tc_combine_mm(acc, y, dis, b, w):
    return pl.pallas_call(
        _combine_mm_body,
        grid=(N // _RB,),
        in_specs=[pl.BlockSpec((NC, _RB, D), lambda i: (0, i, 0)),
                  pl.BlockSpec((_RB, D), lambda i: (i, 0)),
                  pl.BlockSpec((_RB, 16), lambda i: (i, 0)),
                  pl.BlockSpec((1, D), lambda i: (0, 0)),
                  pl.BlockSpec((D, D), lambda i: (0, 0))],
        out_specs=pl.BlockSpec((_RB, D), lambda i: (i, 0)),
        out_shape=jax.ShapeDtypeStruct((N, D), jnp.float32),
    )(acc, y, dis, b, w)


def _pool_body(acc_ref, y2_ref, dis_ref, b_ref, batch_ref, wp_ref, bp_ref,
               out_ref, sums_sc, cnts_sc):
    i = pl.program_id(0)

    @pl.when(i == 0)
    def _():
        sums_sc[...] = jnp.zeros_like(sums_sc)
        cnts_sc[...] = jnp.zeros_like(cnts_sc)

    dis = dis_ref[:, 0:1]
    h2 = dis * (acc_ref[0] + acc_ref[1] + y2_ref[...]) + b_ref[...]
    gids = lax.broadcasted_iota(jnp.int32, (G, _RB), 0)
    oh = (batch_ref[0] == gids).astype(jnp.float32)
    sums_sc[...] += jnp.dot(oh, h2, preferred_element_type=jnp.float32)
    cnts_sc[...] += jnp.sum(oh, axis=1, keepdims=True)

    @pl.when(i == N // _RB - 1)
    def _():
        pooled = sums_sc[...] / jnp.maximum(cnts_sc[:, 0:1], 1.0)
        out_ref[...] = jnp.dot(pooled, wp_ref[...],
                               preferred_element_type=jnp.float32) + bp_ref[...]


def _tc_pool(acc, y2, dis, b, batch3d, wp_pad, bp_pad):
    return pl.pallas_call(
        _pool_body,
        grid=(N // _RB,),
        in_specs=[pl.BlockSpec((NC, _RB, D), lambda i: (0, i, 0)),
                  pl.BlockSpec((_RB, D), lambda i: (i, 0)),
                  pl.BlockSpec((_RB, 16), lambda i: (i, 0)),
                  pl.BlockSpec((1, D), lambda i: (0, 0)),
                  pl.BlockSpec((1, 1, _RB), lambda i: (i, 0, 0)),
                  pl.BlockSpec((D, D), lambda i: (0, 0)),
                  pl.BlockSpec((1, D), lambda i: (0, 0))],
        out_specs=pl.BlockSpec((G, D), lambda i: (0, 0)),
        out_shape=jax.ShapeDtypeStruct((G, D), jnp.float32),
        scratch_shapes=[pltpu.VMEM((G, D), jnp.float32),
                        pltpu.VMEM((G, 1), jnp.float32)],
    )(acc, y2, dis, b, batch3d, wp_pad, bp_pad)


def kernel(x, edge_index, batch, W1, b1, W2, b2, Wp, bp):
    src = edge_index[0].astype(jnp.int32)
    dst = edge_index[1].astype(jnp.int32)
    pad = T * EPT - E
    pad_dst = N + (jnp.arange(pad, dtype=jnp.int32) % (NPAD - N))
    pad_src = jnp.arange(pad, dtype=jnp.int32) % N
    src_t = jnp.concatenate([src, pad_src]).reshape(T, CPT, CHUNK)
    dst_t = jnp.concatenate([dst, pad_dst]).reshape(T, CPT, CHUNK)

    znd = jnp.zeros((NPAD, D), jnp.float32)
    degh = _sc_hist_kernel()(dst_t)
    xw1 = _tc_matmul(x, W1)
    dis, y1 = _tc_dis_y(degh, xw1)

    acc1 = _sc_scatter_kernel()(y1, src_t, dst_t, znd)
    y2 = _tc_combine_mm(acc1, y1, dis, b1.reshape(1, D), W2)

    acc2 = _sc_scatter_kernel()(y2, src_t, dst_t, znd)
    batch3d = batch.astype(jnp.int32).reshape(N // _RB, 1, _RB)
    wp_pad = jnp.pad(Wp, ((0, 0), (0, D - Wp.shape[1])))
    bp_pad = jnp.pad(bp, (0, D - bp.shape[0])).reshape(1, D)
    out_pad = _tc_pool(acc2, y2, dis, b2.reshape(1, D),
                       batch3d, wp_pad, bp_pad)
    return out_pad[:, :Wp.shape[1]]


# speedup vs baseline: 3.3511x; 1.3466x over previous
import functools

import jax
import jax.numpy as jnp
from jax import lax
from jax.experimental import pallas as pl
from jax.experimental.pallas import tpu as pltpu
from jax.experimental.pallas import tpu_sc as plsc

N = 10000
E = 320000
D = 128
G = 128
NC, NS = 2, 16
T = NC * NS
CHUNK = 128
NBUF = 2
NPH = 2
CPT = 80
HPT = CPT // NPH
EPT = CPT * CHUNK
NPAD = 10240
RPT = NPAD // NS

@functools.cache
def _mesh():
    return plsc.VectorSubcoreMesh(core_axis_name="c", subcore_axis_name="s")


def _fill(buf, value):
    @pl.loop(0, buf.shape[0])
    def _(i):
        @pl.loop(0, D // 16)
        def _(k):
            buf[i, pl.ds(k * 16, 16)] = jnp.full((16,), value, jnp.float32)


def _zero_shared(zbuf, shared, s):
    @pl.loop(0, RPT // CHUNK)
    def _(r):
        pltpu.sync_copy(zbuf, shared.at[pl.ds(s * RPT + r * CHUNK, CHUNK)])


@functools.cache
def _sc_hist_kernel():
    return pl.kernel(
        _sc_hist_body,
        mesh=_mesh(),
        out_type=jax.ShapeDtypeStruct((NC, NPAD, D), jnp.float32),
        scratch_types=[
            pltpu.VMEM((CPT, CHUNK), jnp.int32),
            pltpu.VMEM((CHUNK, D), jnp.float32),
            pltpu.VMEM_SHARED((NPAD, D), jnp.float32),
        ],
    )


def _sc_hist_body(dst_hbm, deg_hbm, dst_v, ones_v, hist_sh):
    c = lax.axis_index("c")
    s = lax.axis_index("s")
    wid = s * NC + c
    pltpu.sync_copy(dst_hbm.at[wid], dst_v)
    _fill(ones_v, 0.0)
    _zero_shared(ones_v, hist_sh, s)
    _fill(ones_v, 1.0)
    plsc.subcore_barrier()

    @pl.loop(0, CPT)
    def _(j):
        pltpu.sync_copy(ones_v, hist_sh.at[dst_v.at[j]], add=True)

    plsc.subcore_barrier()
    pltpu.sync_copy(hist_sh.at[pl.ds(s * RPT, RPT)],
                    deg_hbm.at[c].at[pl.ds(s * RPT, RPT)])


@functools.cache
def _sc_scatter_kernel():
    return pl.kernel(
        _sc_scatter_body,
        mesh=_mesh(),
        out_type=jax.ShapeDtypeStruct((NC, NPAD, D), jnp.float32),
        scratch_types=(
            [pltpu.VMEM((HPT, CHUNK), jnp.int32),
             pltpu.VMEM((HPT, CHUNK), jnp.int32)]
            + [pltpu.VMEM((CHUNK, D), jnp.float32)] * NBUF
            + [pltpu.VMEM_SHARED((NPAD, D), jnp.float32)]
            + [pltpu.SemaphoreType.DMA] * NBUF
        ),
    )


def _sc_scatter_body(y_hbm, src_hbm, dst_hbm, znd_hbm, acc_hbm,
                     src_v, dst_v, *rest):
    bufs = rest[:NBUF]
    acc_sh = rest[NBUF]
    sems = rest[NBUF + 1:]
    c = lax.axis_index("c")
    s = lax.axis_index("s")
    wid = s * NC + c
    pltpu.sync_copy(znd_hbm.at[pl.ds(s * RPT, RPT)],
                    acc_sh.at[pl.ds(s * RPT, RPT)])
    plsc.subcore_barrier()

    def gather(j, b):
        pltpu.make_async_copy(y_hbm.at[src_v.at[j]], bufs[b], sems[b]).start()

    def wait_scatter(j, b):
        pltpu.make_async_copy(y_hbm.at[src_v.at[j]], bufs[b], sems[b]).wait()
        pltpu.sync_copy(bufs[b], acc_sh.at[dst_v.at[j]], add=True)

    for ph in range(NPH):
        pltpu.sync_copy(src_hbm.at[wid].at[pl.ds(ph * HPT, HPT)], src_v)
        pltpu.sync_copy(dst_hbm.at[wid].at[pl.ds(ph * HPT, HPT)], dst_v)
        for b in range(NBUF):
            gather(b, b)

        @pl.loop(0, HPT // NBUF - 1)
        def _(p):
            base = p * NBUF
            for b in range(NBUF):
                wait_scatter(base + b, b)
                gather(base + NBUF + b, b)

        for b in range(NBUF):
            wait_scatter(HPT - NBUF + b, b)

    plsc.subcore_barrier()
    pltpu.sync_copy(acc_sh.at[pl.ds(s * RPT, RPT)],
                    acc_hbm.at[c].at[pl.ds(s * RPT, RPT)])


_RB = 1000


def _mm_body(x_ref, w_ref, o_ref):
    o_ref[...] = jnp.dot(x_ref[...], w_ref[...],
                         preferred_element_type=jnp.float32)


def _tc_matmul(x, w):
    return pl.pallas_call(
        _mm_body,
        grid=(N // _RB,),
        in_specs=[pl.BlockSpec((_RB, D), lambda i: (i, 0)),
                  pl.BlockSpec((D, D), lambda i: (0, 0))],
        out_specs=pl.BlockSpec((_RB, D), lambda i: (i, 0)),
        out_shape=jax.ShapeDtypeStruct((N, D), jnp.float32),
    )(x, w)


def _dis_y_body(degh_ref, xw_ref, dis_ref, y_ref):
    deg = degh_ref[0] + degh_ref[1] + 1.0
    dis = lax.rsqrt(deg)
    dis_ref[...] = dis[:, 0:16]
    y_ref[...] = xw_ref[...] * dis[:, 0:1]


def _tc_dis_y(degh, xw):
    return pl.pallas_call(
        _dis_y_body,
        grid=(N // _RB,),
        in_specs=[pl.BlockSpec((NC, _RB, D), lambda i: (0, i, 0)),
                  pl.BlockSpec((_RB, D), lambda i: (i, 0))],
        out_specs=[pl.BlockSpec((_RB, 16), lambda i: (i, 0)),
                   pl.BlockSpec((_RB, D), lambda i: (i, 0))],
        out_shape=[jax.ShapeDtypeStruct((N, 16), jnp.float32),
                   jax.ShapeDtypeStruct((N, D), jnp.float32)],
    )(degh, xw)


def _combine_mm_body(acc_ref, y_ref, dis_ref, b_ref, w_ref, y2_ref):
    dis = dis_ref[:, 0:1]
    h = dis * (acc_ref[0] + acc_ref[1] + y_ref[...]) + b_ref[...]
    h = jnp.maximum(h, 0.0)
    y2_ref[...] = jnp.dot(h, w_ref[...],
                          preferred_element_type=jnp.float32) * dis


def _tc_combine_mm(acc, y, dis, b, w):
    return pl.pallas_call(
        _combine_mm_body,
        grid=(N // _RB,),
        in_specs=[pl.BlockSpec((NC, _RB, D), lambda i: (0, i, 0)),
                  pl.BlockSpec((_RB, D), lambda i: (i, 0)),
                  pl.BlockSpec((_RB, 16), lambda i: (i, 0)),
                  pl.BlockSpec((1, D), lambda i: (0, 0)),
                  pl.BlockSpec((D, D), lambda i: (0, 0))],
        out_specs=pl.BlockSpec((_RB, D), lambda i: (i, 0)),
        out_shape=jax.ShapeDtypeStruct((N, D), jnp.float32),
    )(acc, y, dis, b, w)


def _pool_body(acc_ref, y2_ref, dis_ref, b_ref, batch_ref, wp_ref, bp_ref,
               out_ref, sums_sc, cnts_sc):
    i = pl.program_id(0)

    @pl.when(i == 0)
    def _():
        sums_sc[...] = jnp.zeros_like(sums_sc)
        cnts_sc[...] = jnp.zeros_like(cnts_sc)

    dis = dis_ref[:, 0:1]
    h2 = dis * (acc_ref[0] + acc_ref[1] + y2_ref[...]) + b_ref[...]
    gids = lax.broadcasted_iota(jnp.int32, (G, _RB), 0)
    oh = (batch_ref[0] == gids).astype(jnp.float32)
    sums_sc[...] += jnp.dot(oh, h2, preferred_element_type=jnp.float32)
    cnts_sc[...] += jnp.sum(oh, axis=1, keepdims=True)

    @pl.when(i == N // _RB - 1)
    def _():
        pooled = sums_sc[...] / jnp.maximum(cnts_sc[:, 0:1], 1.0)
        out_ref[...] = jnp.dot(pooled, wp_ref[...],
                               preferred_element_type=jnp.float32) + bp_ref[...]


def _tc_pool(acc, y2, dis, b, batch3d, wp_pad, bp_pad):
    return pl.pallas_call(
        _pool_body,
        grid=(N // _RB,),
        in_specs=[pl.BlockSpec((NC, _RB, D), lambda i: (0, i, 0)),
                  pl.BlockSpec((_RB, D), lambda i: (i, 0)),
                  pl.BlockSpec((_RB, 16), lambda i: (i, 0)),
                  pl.BlockSpec((1, D), lambda i: (0, 0)),
                  pl.BlockSpec((1, 1, _RB), lambda i: (i, 0, 0)),
                  pl.BlockSpec((D, D), lambda i: (0, 0)),
                  pl.BlockSpec((1, D), lambda i: (0, 0))],
        out_specs=pl.BlockSpec((G, D), lambda i: (0, 0)),
        out_shape=jax.ShapeDtypeStruct((G, D), jnp.float32),
        scratch_shapes=[pltpu.VMEM((G, D), jnp.float32),
                        pltpu.VMEM((G, 1), jnp.float32)],
    )(acc, y2, dis, b, batch3d, wp_pad, bp_pad)


def kernel(x, edge_index, batch, W1, b1, W2, b2, Wp, bp):
    src = edge_index[0].astype(jnp.int32)
    dst = edge_index[1].astype(jnp.int32)
    pad = T * EPT - E
    pad_dst = N + (jnp.arange(pad, dtype=jnp.int32) % (NPAD - N))
    pad_src = jnp.arange(pad, dtype=jnp.int32) % N
    src_t = jnp.concatenate([src, pad_src]).reshape(T, CPT, CHUNK)
    dst_t = jnp.concatenate([dst, pad_dst]).reshape(T, CPT, CHUNK)

    znd = jnp.zeros((NPAD, D), jnp.float32)
    degh = _sc_hist_kernel()(dst_t)
    xw1 = _tc_matmul(x, W1)
    dis, y1 = _tc_dis_y(degh, xw1)

    acc1 = _sc_scatter_kernel()(y1, src_t, dst_t, znd)
    y2 = _tc_combine_mm(acc1, y1, dis, b1.reshape(1, D), W2)

    acc2 = _sc_scatter_kernel()(y2, src_t, dst_t, znd)
    batch3d = batch.astype(jnp.int32).reshape(N // _RB, 1, _RB)
    wp_pad = jnp.pad(Wp, ((0, 0), (0, D - Wp.shape[1])))
    bp_pad = jnp.pad(bp, (0, D - bp.shape[0])).reshape(1, D)
    out_pad = _tc_pool(acc2, y2, dis, b2.reshape(1, D),
                       batch3d, wp_pad, bp_pad)
    return out_pad[:, :Wp.shape[1]]
